# speedup vs baseline: 99.8101x; 99.8101x over previous
import functools

import jax
import jax.numpy as jnp
from jax.experimental import pallas as pl
from jax.experimental.pallas import tpu as pltpu

TIMESTEPS = 32
MAX_LATENCY = 1.0


def _latency_kernel(data_ref, out_ref, minmax_ref, *, block_n: int):
    i = pl.program_id(0)

    @pl.when(i == 0)
    def _compute_minmax():
        x = data_ref[...]
        minmax_ref[0] = jnp.min(x)
        minmax_ref[1] = jnp.max(x)

    dmin = minmax_ref[0]
    dmax = minmax_ref[1]
    has_range = dmax > dmin
    denom = jnp.where(has_range, dmax - dmin, jnp.float32(1.0))

    x = data_ref[:, pl.ds(i * block_n, block_n)]
    normalized = jnp.where(has_range, (x - dmin) / denom, jnp.float32(0.5))
    latencies = (1.0 - normalized) * MAX_LATENCY
    times = jnp.clip((latencies * (TIMESTEPS - 1)).astype(jnp.int32),
                     0, TIMESTEPS - 1)

    t_iota = jax.lax.broadcasted_iota(
        jnp.int32, (out_ref.shape[0], TIMESTEPS, block_n), 1)
    out_ref[...] = (t_iota == times[:, None, :]).astype(jnp.float32)


def kernel(data):
    squeeze = False
    if data.ndim == 1:
        data = data[None, :]
        squeeze = True
    batch = data.shape[0]
    feat_shape = data.shape[1:]
    flat = data.reshape(batch, -1)
    n = flat.shape[1]

    block_n = 512
    while n % block_n:
        block_n //= 2
    grid = n // block_n

    out = pl.pallas_call(
        functools.partial(_latency_kernel, block_n=block_n),
        grid=(grid,),
        in_specs=[pl.BlockSpec((batch, n), lambda i: (0, 0))],
        out_specs=pl.BlockSpec((batch, TIMESTEPS, block_n),
                               lambda i: (0, 0, i)),
        out_shape=jax.ShapeDtypeStruct((batch, TIMESTEPS, n), jnp.float32),
        scratch_shapes=[pltpu.SMEM((2,), jnp.float32)],
    )(flat)

    out = out.reshape(batch, TIMESTEPS, *feat_shape)
    if squeeze:
        out = out[0]
    return out
